# baseline (device time: 12363 ns/iter reference)
import jax
import jax.numpy as jnp
from jax import lax
from jax.experimental import pallas as pl
from jax.experimental.pallas import tpu as pltpu

N_DEV = 4


def kernel(x, w_mat):
    m_g, kb = x.shape
    k_g, n = w_mat.shape
    mb = m_g // N_DEV
    nh = n // 2

    def body(x_ref, w_ref, out_ref, sb_ref, xg_ref, send_sems, recv_sems):
        my = lax.axis_index("i")

        for o in (2, 1, 3):
            dst = (my + o) % N_DEV
            sb_ref[dst] = x_ref[pl.ds(dst * mb, mb), :].astype(jnp.bfloat16)

        barrier_sem = pltpu.get_barrier_semaphore()
        for o in range(1, N_DEV):
            peer = (my + o) % N_DEV
            pl.semaphore_signal(
                barrier_sem, inc=1,
                device_id=(peer,), device_id_type=pl.DeviceIdType.MESH,
            )
        pl.semaphore_wait(barrier_sem, N_DEV - 1)

        khb = kb // 2
        sends = []
        for o in (2, 1, 3):
            dst = (my + o) % N_DEV
            for h in range(2):
                rdma = pltpu.make_async_remote_copy(
                    src_ref=sb_ref.at[dst, :, pl.ds(h * khb, khb)],
                    dst_ref=xg_ref.at[my, :, pl.ds(h * khb, khb)],
                    send_sem=send_sems.at[o - 1, h],
                    recv_sem=recv_sems.at[my, h],
                    device_id=(dst,),
                    device_id_type=pl.DeviceIdType.MESH,
                )
                rdma.start()
                sends.append(rdma)

        xl = x_ref[pl.ds(my * mb, mb), :]
        wl = w_ref[pl.ds(my * kb, kb), :]
        acc0 = jnp.dot(xl, wl[:, :nh], preferred_element_type=jnp.float32)
        acc1 = jnp.dot(xl, wl[:, nh:], preferred_element_type=jnp.float32)

        for o in (1, 3, 2):
            src = (my + o) % N_DEV
            for h in range(2):
                recv = pltpu.make_async_remote_copy(
                    src_ref=sb_ref.at[src, :, pl.ds(h * khb, khb)],
                    dst_ref=xg_ref.at[src, :, pl.ds(h * khb, khb)],
                    send_sem=send_sems.at[o - 1, h],
                    recv_sem=recv_sems.at[src, h],
                    device_id=(src,),
                    device_id_type=pl.DeviceIdType.MESH,
                )
                recv.wait_recv()
                xr = xg_ref[src, :, pl.ds(h * khb, khb)].astype(jnp.float32)
                wr = w_ref[pl.ds(src * kb + h * khb, khb), :]
                acc0 = acc0 + jnp.dot(xr, wr[:, :nh], preferred_element_type=jnp.float32)
                acc1 = acc1 + jnp.dot(xr, wr[:, nh:], preferred_element_type=jnp.float32)

        c = 0.7978845608028654
        out_ref[:, :nh] = 0.5 * acc0 * (
            1.0 + jnp.tanh(c * (acc0 + 0.044715 * acc0 * acc0 * acc0)))
        out_ref[:, nh:] = 0.5 * acc1 * (
            1.0 + jnp.tanh(c * (acc1 + 0.044715 * acc1 * acc1 * acc1)))

        for rdma in sends:
            rdma.wait_send()

    return pl.pallas_call(
        body,
        out_shape=jax.ShapeDtypeStruct((mb, n), jnp.float32),
        in_specs=[
            pl.BlockSpec(memory_space=pltpu.VMEM),
            pl.BlockSpec(memory_space=pltpu.VMEM),
        ],
        out_specs=pl.BlockSpec(memory_space=pltpu.VMEM),
        scratch_shapes=[
            pltpu.VMEM((N_DEV, mb, kb), jnp.bfloat16),
            pltpu.VMEM((N_DEV, mb, kb), jnp.bfloat16),
            pltpu.SemaphoreType.DMA((N_DEV - 1, 2)),
            pltpu.SemaphoreType.DMA((N_DEV, 2)),
        ],
        compiler_params=pltpu.CompilerParams(collective_id=0),
    )(x, w_mat)
